# Initial kernel scaffold; baseline (speedup 1.0000x reference)
import functools

import jax
import jax.numpy as jnp
from jax.experimental import pallas as pl
from jax.experimental.pallas import tpu as pltpu

HEADS = 5
ROW_BLOCK = 2500


def _leaky(x, s=0.01):
    return jnp.where(x >= 0, x, s * x)


def _softplus(x):
    return jnp.maximum(x, 0.0) + jnp.log1p(jnp.exp(-jnp.abs(x)))


def _linear_body(x_ref, w_ref, b_ref, o_ref, *, act):
    y = jnp.dot(x_ref[...], w_ref[...].T, preferred_element_type=jnp.float32)
    y = y + b_ref[...]
    if act is not None:
        y = act(y)
    o_ref[...] = y


def _linear_scaled_body(x_ref, w_ref, b_ref, s_ref, o_ref, *, act):
    y = jnp.dot(x_ref[...], w_ref[...].T, preferred_element_type=jnp.float32)
    y = y + b_ref[...]
    if act is not None:
        y = act(y)
    o_ref[...] = y * s_ref[...]


def tc_linear(x, w, b, act=None, row_scale=None):
    n, k = x.shape
    f = w.shape[0]
    bn = ROW_BLOCK
    assert n % bn == 0
    b2 = b.reshape(1, f)
    if row_scale is None:
        body = functools.partial(_linear_body, act=act)
        return pl.pallas_call(
            body,
            grid=(n // bn,),
            in_specs=[
                pl.BlockSpec((bn, k), lambda i: (i, 0)),
                pl.BlockSpec((f, k), lambda i: (0, 0)),
                pl.BlockSpec((1, f), lambda i: (0, 0)),
            ],
            out_specs=pl.BlockSpec((bn, f), lambda i: (i, 0)),
            out_shape=jax.ShapeDtypeStruct((n, f), jnp.float32),
        )(x, w, b2)
    body = functools.partial(_linear_scaled_body, act=act)
    return pl.pallas_call(
        body,
        grid=(n // bn,),
        in_specs=[
            pl.BlockSpec((bn, k), lambda i: (i, 0)),
            pl.BlockSpec((f, k), lambda i: (0, 0)),
            pl.BlockSpec((1, f), lambda i: (0, 0)),
            pl.BlockSpec((bn, 1), lambda i: (i, 0)),
        ],
        out_specs=pl.BlockSpec((bn, f), lambda i: (i, 0)),
        out_shape=jax.ShapeDtypeStruct((n, f), jnp.float32),
    )(x, w, b2, row_scale.reshape(n, 1))


def _stats_body(x_ref, o_ref):
    i = pl.program_id(0)

    @pl.when(i == 0)
    def _():
        o_ref[...] = jnp.zeros_like(o_ref)

    xx = x_ref[...]
    s = jnp.sum(xx, axis=0, keepdims=True)
    ss = jnp.sum(xx * xx, axis=0, keepdims=True)
    pad = jnp.zeros((6, xx.shape[1]), jnp.float32)
    o_ref[...] += jnp.concatenate([s, ss, pad], axis=0)


def tc_colstats(x):
    n, f = x.shape
    bn = ROW_BLOCK
    return pl.pallas_call(
        _stats_body,
        grid=(n // bn,),
        in_specs=[pl.BlockSpec((bn, f), lambda i: (i, 0))],
        out_specs=pl.BlockSpec((8, f), lambda i: (0, 0)),
        out_shape=jax.ShapeDtypeStruct((8, f), jnp.float32),
    )(x)


def _bn_leaky_body(x_ref, st_ref, g_ref, b_ref, o_ref, *, n, eps):
    mu = st_ref[0:1, :] / n
    var = st_ref[1:2, :] / n - mu * mu
    y = (x_ref[...] - mu) / jnp.sqrt(var + eps) * g_ref[...] + b_ref[...]
    o_ref[...] = _leaky(y)


def tc_bn_leaky(x, stats, g, b, eps=1e-5):
    n, f = x.shape
    bn = ROW_BLOCK
    body = functools.partial(_bn_leaky_body, n=float(n), eps=eps)
    return pl.pallas_call(
        body,
        grid=(n // bn,),
        in_specs=[
            pl.BlockSpec((bn, f), lambda i: (i, 0)),
            pl.BlockSpec((8, f), lambda i: (0, 0)),
            pl.BlockSpec((1, f), lambda i: (0, 0)),
            pl.BlockSpec((1, f), lambda i: (0, 0)),
        ],
        out_specs=pl.BlockSpec((bn, f), lambda i: (i, 0)),
        out_shape=jax.ShapeDtypeStruct((n, f), jnp.float32),
    )(x, stats, g.reshape(1, f), b.reshape(1, f))


def _gcn_post_body(p_ref, s_ref, b_ref, o_ref):
    agg = p_ref[0] + p_ref[1]
    o_ref[...] = agg * s_ref[...] + b_ref[...]


def tc_gcn_post(partials, dinv, b):
    _, n, f = partials.shape
    bn = ROW_BLOCK
    return pl.pallas_call(
        _gcn_post_body,
        grid=(n // bn,),
        in_specs=[
            pl.BlockSpec((2, bn, f), lambda i: (0, i, 0)),
            pl.BlockSpec((bn, 1), lambda i: (i, 0)),
            pl.BlockSpec((1, f), lambda i: (0, 0)),
        ],
        out_specs=pl.BlockSpec((bn, f), lambda i: (i, 0)),
        out_shape=jax.ShapeDtypeStruct((n, f), jnp.float32),
    )(partials, dinv.reshape(n, 1), b.reshape(1, f))


def _gat_post_body(num_ref, den_ref, b_ref, o_ref, *, heads):
    den = den_ref[...]
    acc = None
    for h in range(heads):
        nh = num_ref[h]
        dh = den[:, h:h + 1]
        t = nh / (dh + 1e-16)
        acc = t if acc is None else acc + t
    o_ref[...] = acc * (1.0 / heads) + b_ref[...]


def tc_gat_post(num, den, b):
    heads, n, fh = num.shape
    bn = ROW_BLOCK
    body = functools.partial(_gat_post_body, heads=heads)
    return pl.pallas_call(
        body,
        grid=(n // bn,),
        in_specs=[
            pl.BlockSpec((heads, bn, fh), lambda i: (0, i, 0)),
            pl.BlockSpec((bn, heads), lambda i: (i, 0)),
            pl.BlockSpec((1, fh), lambda i: (0, 0)),
        ],
        out_specs=pl.BlockSpec((bn, fh), lambda i: (i, 0)),
        out_shape=jax.ShapeDtypeStruct((n, fh), jnp.float32),
    )(num, den, b.reshape(1, fh))


def edge_degree(dst, n):
    return jnp.zeros((n,), jnp.float32).at[dst].add(1.0)


def edge_gather_scatter(g, src, dst, n):
    return jnp.zeros((n, g.shape[1]), jnp.float32).at[dst].add(g[src])


def edge_gat(h, as_, ad_, src, dst, n, heads):
    e = _leaky(as_[src] + ad_[dst], 0.2)
    m = jax.ops.segment_max(e, dst, num_segments=n)
    ex = jnp.exp(e - m[dst])
    den = jax.ops.segment_sum(ex, dst, num_segments=n)
    num = jax.ops.segment_sum(h[src] * ex[:, :, None], dst, num_segments=n)
    return num, den


def _gcn_layer(x, src, dst, w, b, dinv, n):
    g = tc_linear(x, w, jnp.zeros((w.shape[0],), jnp.float32),
                  row_scale=dinv)
    agg = edge_gather_scatter(g, src, dst, n)
    partials = jnp.stack([agg, jnp.zeros_like(agg)])
    return tc_gcn_post(partials, dinv, b)


def _gat_layer(x, src, dst, p, pref, n):
    w = p[pref + '_w']
    fh = w.shape[0] // HEADS
    h = tc_linear(x, w, jnp.zeros((w.shape[0],), jnp.float32))
    a_s, a_d = p[pref + '_as'], p[pref + '_ad']
    as_mat = jax.scipy.linalg.block_diag(*[a_s[k:k + 1, :] for k in range(HEADS)])
    ad_mat = jax.scipy.linalg.block_diag(*[a_d[k:k + 1, :] for k in range(HEADS)])
    zeros5 = jnp.zeros((HEADS,), jnp.float32)
    as_ = tc_linear(h, as_mat, zeros5)
    ad_ = tc_linear(h, ad_mat, zeros5)
    num, den = edge_gat(h.reshape(n, HEADS, fh), as_, ad_, src, dst, n, HEADS)
    num = jnp.transpose(num, (1, 0, 2))
    return tc_gat_post(num, den, p[pref + '_b'])


def kernel(x_in, edge_index, params):
    p = params
    n = x_in.shape[0]
    loop = jnp.arange(n, dtype=edge_index.dtype)
    src = jnp.concatenate([edge_index[0], loop])
    dst = jnp.concatenate([edge_index[1], loop])

    deg = edge_degree(dst, n)
    dinv = jnp.where(deg > 0, jax.lax.rsqrt(deg), 0.0)

    x = tc_linear(x_in, p['nn1_w1'], p['nn1_b1'], act=_softplus)
    x = tc_linear(x, p['nn1_w2'], p['nn1_b2'], act=_softplus)
    x = tc_linear(x, p['nn1_w3'], p['nn1_b3'])
    x = tc_bn_leaky(x, tc_colstats(x), p['bn0_g'], p['bn0_b'])

    x1 = _gcn_layer(x, src, dst, p['gcn1_w'], p['gcn1_b'], dinv, n)
    x1 = _gat_layer(x1, src, dst, p, 'gat1', n)
    x1 = tc_bn_leaky(x1, tc_colstats(x1), p['bn1_g'], p['bn1_b'])

    skip1 = jnp.concatenate([x, x1], axis=1)
    x2 = _gcn_layer(skip1, src, dst, p['gcn2_w'], p['gcn2_b'], dinv, n)
    x2 = _gat_layer(x2, src, dst, p, 'gat2', n)
    x2 = tc_bn_leaky(x2, tc_colstats(x2), p['bn2_g'], p['bn2_b'])

    skip2 = jnp.concatenate([x1, x2], axis=1)
    x3 = _gcn_layer(skip2, src, dst, p['gcn3_w'], p['gcn3_b'], dinv, n)
    x3 = _gat_layer(x3, src, dst, p, 'gat3', n)
    x3 = tc_bn_leaky(x3, tc_colstats(x3), p['bn3_g'], p['bn3_b'])

    xf = jnp.concatenate([x, x3], axis=1)
    hh = tc_linear(xf, p['np_w1'], p['np_b1'], act=_softplus)
    hh = tc_linear(hh, p['np_w2'], p['np_b2'], act=_softplus)
    hh = tc_linear(hh, p['np_w3'], p['np_b3'], act=_softplus)
    logits = tc_linear(hh, p['np_w4'], p['np_b4'])
    probs = 1.0 / (1.0 + jnp.exp(-logits))
    return xf, probs


# speedup vs baseline: 1.0764x; 1.0764x over previous
import functools

import jax
import jax.numpy as jnp
from jax.experimental import pallas as pl
from jax.experimental.pallas import tpu as pltpu

HEADS = 5
ROW_BLOCK = 2000


def _leaky(x, s=0.01):
    return jnp.where(x >= 0, x, s * x)


def _softplus(x):
    return jnp.maximum(x, 0.0) + jnp.log1p(jnp.exp(-jnp.abs(x)))


def _linear_body(x_ref, w_ref, b_ref, o_ref, *, act):
    y = jnp.dot(x_ref[...], w_ref[...].T, preferred_element_type=jnp.float32)
    y = y + b_ref[...]
    if act is not None:
        y = act(y)
    o_ref[...] = y


def _linear_scaled_body(x_ref, w_ref, b_ref, s_ref, o_ref, *, act):
    y = jnp.dot(x_ref[...], w_ref[...].T, preferred_element_type=jnp.float32)
    y = y + b_ref[...]
    if act is not None:
        y = act(y)
    s = jnp.sum(s_ref[...], axis=1, keepdims=True)
    o_ref[...] = y * s


def tc_linear(x, w, b, act=None, row_scale=None):
    n, k = x.shape
    f = w.shape[0]
    bn = ROW_BLOCK
    assert n % bn == 0
    b2 = b.reshape(1, f)
    if row_scale is None:
        body = functools.partial(_linear_body, act=act)
        return pl.pallas_call(
            body,
            grid=(n // bn,),
            in_specs=[
                pl.BlockSpec((bn, k), lambda i: (i, 0)),
                pl.BlockSpec((f, k), lambda i: (0, 0)),
                pl.BlockSpec((1, f), lambda i: (0, 0)),
            ],
            out_specs=pl.BlockSpec((bn, f), lambda i: (i, 0)),
            out_shape=jax.ShapeDtypeStruct((n, f), jnp.float32),
        )(x, w, b2)
    body = functools.partial(_linear_scaled_body, act=act)
    return pl.pallas_call(
        body,
        grid=(n // bn,),
        in_specs=[
            pl.BlockSpec((bn, k), lambda i: (i, 0)),
            pl.BlockSpec((f, k), lambda i: (0, 0)),
            pl.BlockSpec((1, f), lambda i: (0, 0)),
            pl.BlockSpec((bn, 1), lambda i: (i, 0)),
        ],
        out_specs=pl.BlockSpec((bn, f), lambda i: (i, 0)),
        out_shape=jax.ShapeDtypeStruct((n, f), jnp.float32),
    )(x, w, b2, row_scale.reshape(n, 1))


def _stats_body(x_ref, o_ref):
    i = pl.program_id(0)

    @pl.when(i == 0)
    def _():
        o_ref[...] = jnp.zeros_like(o_ref)

    xx = x_ref[...]
    s = jnp.sum(xx, axis=0, keepdims=True)
    ss = jnp.sum(xx * xx, axis=0, keepdims=True)
    pad = jnp.zeros((6, xx.shape[1]), jnp.float32)
    o_ref[...] += jnp.concatenate([s, ss, pad], axis=0)


def tc_colstats(x):
    n, f = x.shape
    bn = ROW_BLOCK
    return pl.pallas_call(
        _stats_body,
        grid=(n // bn,),
        in_specs=[pl.BlockSpec((bn, f), lambda i: (i, 0))],
        out_specs=pl.BlockSpec((8, f), lambda i: (0, 0)),
        out_shape=jax.ShapeDtypeStruct((8, f), jnp.float32),
    )(x)


def _bn_leaky_body(x_ref, st_ref, g_ref, b_ref, o_ref, *, n, eps):
    mu = st_ref[0:1, :] / n
    var = st_ref[1:2, :] / n - mu * mu
    y = (x_ref[...] - mu) / jnp.sqrt(var + eps) * g_ref[...] + b_ref[...]
    o_ref[...] = _leaky(y)


def tc_bn_leaky(x, stats, g, b, eps=1e-5):
    n, f = x.shape
    bn = ROW_BLOCK
    body = functools.partial(_bn_leaky_body, n=float(n), eps=eps)
    return pl.pallas_call(
        body,
        grid=(n // bn,),
        in_specs=[
            pl.BlockSpec((bn, f), lambda i: (i, 0)),
            pl.BlockSpec((8, f), lambda i: (0, 0)),
            pl.BlockSpec((1, f), lambda i: (0, 0)),
            pl.BlockSpec((1, f), lambda i: (0, 0)),
        ],
        out_specs=pl.BlockSpec((bn, f), lambda i: (i, 0)),
        out_shape=jax.ShapeDtypeStruct((n, f), jnp.float32),
    )(x, stats, g.reshape(1, f), b.reshape(1, f))


def _logit_body(x_ref, w_ref, b_ref, o_ref):
    y = jnp.sum(x_ref[...] * w_ref[...], axis=1, keepdims=True) + b_ref[0]
    o_ref[...] = 1.0 / (1.0 + jnp.exp(-y))


def tc_logit_sigmoid(x, w, b):
    n, k = x.shape
    bn = ROW_BLOCK
    return pl.pallas_call(
        _logit_body,
        grid=(n // bn,),
        in_specs=[
            pl.BlockSpec((bn, k), lambda i: (i, 0)),
            pl.BlockSpec((1, k), lambda i: (0, 0)),
            pl.BlockSpec(memory_space=pltpu.SMEM),
        ],
        out_specs=pl.BlockSpec((bn, 1), lambda i: (i, 0)),
        out_shape=jax.ShapeDtypeStruct((n, 1), jnp.float32),
    )(x, w, b)


def _gcn_post_body(p_ref, s_ref, b_ref, o_ref):
    agg = p_ref[0] + p_ref[1]
    s = jnp.sum(s_ref[...], axis=1, keepdims=True)
    o_ref[...] = agg * s + b_ref[...]


def tc_gcn_post(partials, dinv, b):
    _, n, f = partials.shape
    bn = ROW_BLOCK
    return pl.pallas_call(
        _gcn_post_body,
        grid=(n // bn,),
        in_specs=[
            pl.BlockSpec((2, bn, f), lambda i: (0, i, 0)),
            pl.BlockSpec((bn, 1), lambda i: (i, 0)),
            pl.BlockSpec((1, f), lambda i: (0, 0)),
        ],
        out_specs=pl.BlockSpec((bn, f), lambda i: (i, 0)),
        out_shape=jax.ShapeDtypeStruct((n, f), jnp.float32),
    )(partials, dinv.reshape(n, 1), b.reshape(1, f))


def _gat_post_body(num_ref, den_ref, b_ref, o_ref, *, heads):
    acc = None
    for h in range(heads):
        nh = num_ref[h]
        dh = jnp.sum(den_ref[h], axis=1, keepdims=True)
        t = nh / (dh + 1e-16)
        acc = t if acc is None else acc + t
    o_ref[...] = acc * (1.0 / heads) + b_ref[...]


def tc_gat_post(num, den_t, b):
    heads, n, fh = num.shape
    bn = ROW_BLOCK
    body = functools.partial(_gat_post_body, heads=heads)
    return pl.pallas_call(
        body,
        grid=(n // bn,),
        in_specs=[
            pl.BlockSpec((heads, bn, fh), lambda i: (0, i, 0)),
            pl.BlockSpec((heads, bn, 1), lambda i: (0, i, 0)),
            pl.BlockSpec((1, fh), lambda i: (0, 0)),
        ],
        out_specs=pl.BlockSpec((bn, fh), lambda i: (i, 0)),
        out_shape=jax.ShapeDtypeStruct((n, fh), jnp.float32),
    )(num, den_t, b.reshape(1, fh))


def edge_degree(dst, n):
    return jnp.zeros((n,), jnp.float32).at[dst].add(1.0)


def edge_gather_scatter(g, src, dst, n):
    return jnp.zeros((n, g.shape[1]), jnp.float32).at[dst].add(g[src])


def edge_gat(h, as_, ad_, src, dst, n, heads):
    e = _leaky(as_[src] + ad_[dst], 0.2)
    m = jax.ops.segment_max(e, dst, num_segments=n)
    ex = jnp.exp(e - m[dst])
    den = jax.ops.segment_sum(ex, dst, num_segments=n)
    num = jax.ops.segment_sum(h[src] * ex[:, :, None], dst, num_segments=n)
    return num, den


def _gcn_layer(x, src, dst, w, b, dinv, n):
    g = tc_linear(x, w, jnp.zeros((w.shape[0],), jnp.float32),
                  row_scale=dinv)
    agg = edge_gather_scatter(g, src, dst, n)
    partials = jnp.stack([agg, jnp.zeros_like(agg)])
    return tc_gcn_post(partials, dinv, b)


def _gat_layer(x, src, dst, p, pref, n):
    w = p[pref + '_w']
    fh = w.shape[0] // HEADS
    h = tc_linear(x, w, jnp.zeros((w.shape[0],), jnp.float32))
    a_s, a_d = p[pref + '_as'], p[pref + '_ad']
    as_mat = jax.scipy.linalg.block_diag(*[a_s[k:k + 1, :] for k in range(HEADS)])
    ad_mat = jax.scipy.linalg.block_diag(*[a_d[k:k + 1, :] for k in range(HEADS)])
    zeros5 = jnp.zeros((HEADS,), jnp.float32)
    as_ = tc_linear(h, as_mat, zeros5)
    ad_ = tc_linear(h, ad_mat, zeros5)
    num, den = edge_gat(h.reshape(n, HEADS, fh), as_, ad_, src, dst, n, HEADS)
    num = jnp.transpose(num, (1, 0, 2))
    den_t = jnp.transpose(den, (1, 0)).reshape(HEADS, n, 1)
    return tc_gat_post(num, den_t, p[pref + '_b'])


def kernel(x_in, edge_index, params):
    p = params
    n = x_in.shape[0]
    loop = jnp.arange(n, dtype=edge_index.dtype)
    src = jnp.concatenate([edge_index[0], loop])
    dst = jnp.concatenate([edge_index[1], loop])

    deg = edge_degree(dst, n)
    dinv = jnp.where(deg > 0, jax.lax.rsqrt(deg), 0.0)

    x = tc_linear(x_in, p['nn1_w1'], p['nn1_b1'], act=_softplus)
    x = tc_linear(x, p['nn1_w2'], p['nn1_b2'], act=_softplus)
    x = tc_linear(x, p['nn1_w3'], p['nn1_b3'])
    x = tc_bn_leaky(x, tc_colstats(x), p['bn0_g'], p['bn0_b'])

    x1 = _gcn_layer(x, src, dst, p['gcn1_w'], p['gcn1_b'], dinv, n)
    x1 = _gat_layer(x1, src, dst, p, 'gat1', n)
    x1 = tc_bn_leaky(x1, tc_colstats(x1), p['bn1_g'], p['bn1_b'])

    skip1 = jnp.concatenate([x, x1], axis=1)
    x2 = _gcn_layer(skip1, src, dst, p['gcn2_w'], p['gcn2_b'], dinv, n)
    x2 = _gat_layer(x2, src, dst, p, 'gat2', n)
    x2 = tc_bn_leaky(x2, tc_colstats(x2), p['bn2_g'], p['bn2_b'])

    skip2 = jnp.concatenate([x1, x2], axis=1)
    x3 = _gcn_layer(skip2, src, dst, p['gcn3_w'], p['gcn3_b'], dinv, n)
    x3 = _gat_layer(x3, src, dst, p, 'gat3', n)
    x3 = tc_bn_leaky(x3, tc_colstats(x3), p['bn3_g'], p['bn3_b'])

    xf = jnp.concatenate([x, x3], axis=1)
    hh = tc_linear(xf, p['np_w1'], p['np_b1'], act=_softplus)
    hh = tc_linear(hh, p['np_w2'], p['np_b2'], act=_softplus)
    hh = tc_linear(hh, p['np_w3'], p['np_b3'], act=_softplus)
    probs = tc_logit_sigmoid(hh, p['np_w4'], p['np_b4'])
    return xf, probs


# speedup vs baseline: 4.4003x; 4.0881x over previous
import functools

import jax
import jax.numpy as jnp
from jax import lax
from jax.experimental import pallas as pl
from jax.experimental.pallas import tpu as pltpu
from jax.experimental.pallas import tpu_sc as plsc

HEADS = 5
ROW_BLOCK = 2000

N_NODES = 10000
N_ACC = N_NODES + 16
NUM_TILES = 32
CHUNK = 128
TILE_CHUNKS = 164
EDGES_PAD = 16 * TILE_CHUNKS * CHUNK


def _leaky(x, s=0.01):
    return jnp.where(x >= 0, x, s * x)


def _softplus(x):
    return jnp.maximum(x, 0.0) + jnp.log1p(jnp.exp(-jnp.abs(x)))


def _linear_body(x_ref, w_ref, b_ref, o_ref, *, act):
    y = jnp.dot(x_ref[...], w_ref[...].T, preferred_element_type=jnp.float32)
    y = y + b_ref[...]
    if act is not None:
        y = act(y)
    o_ref[...] = y


def _linear_scaled_body(x_ref, w_ref, b_ref, s_ref, o_ref, *, act):
    y = jnp.dot(x_ref[...], w_ref[...].T, preferred_element_type=jnp.float32)
    y = y + b_ref[...]
    if act is not None:
        y = act(y)
    s = jnp.sum(s_ref[...], axis=1, keepdims=True)
    o_ref[...] = y * s


def tc_linear(x, w, b, act=None, row_scale=None):
    n, k = x.shape
    f = w.shape[0]
    bn = ROW_BLOCK
    assert n % bn == 0
    b2 = b.reshape(1, f)
    if row_scale is None:
        body = functools.partial(_linear_body, act=act)
        return pl.pallas_call(
            body,
            grid=(n // bn,),
            in_specs=[
                pl.BlockSpec((bn, k), lambda i: (i, 0)),
                pl.BlockSpec((f, k), lambda i: (0, 0)),
                pl.BlockSpec((1, f), lambda i: (0, 0)),
            ],
            out_specs=pl.BlockSpec((bn, f), lambda i: (i, 0)),
            out_shape=jax.ShapeDtypeStruct((n, f), jnp.float32),
        )(x, w, b2)
    body = functools.partial(_linear_scaled_body, act=act)
    return pl.pallas_call(
        body,
        grid=(n // bn,),
        in_specs=[
            pl.BlockSpec((bn, k), lambda i: (i, 0)),
            pl.BlockSpec((f, k), lambda i: (0, 0)),
            pl.BlockSpec((1, f), lambda i: (0, 0)),
            pl.BlockSpec((bn, 1), lambda i: (i, 0)),
        ],
        out_specs=pl.BlockSpec((bn, f), lambda i: (i, 0)),
        out_shape=jax.ShapeDtypeStruct((n, f), jnp.float32),
    )(x, w, b2, row_scale.reshape(n, 1))


def _stats_body(x_ref, o_ref):
    i = pl.program_id(0)

    @pl.when(i == 0)
    def _():
        o_ref[...] = jnp.zeros_like(o_ref)

    xx = x_ref[...]
    s = jnp.sum(xx, axis=0, keepdims=True)
    ss = jnp.sum(xx * xx, axis=0, keepdims=True)
    pad = jnp.zeros((6, xx.shape[1]), jnp.float32)
    o_ref[...] += jnp.concatenate([s, ss, pad], axis=0)


def tc_colstats(x):
    n, f = x.shape
    bn = ROW_BLOCK
    return pl.pallas_call(
        _stats_body,
        grid=(n // bn,),
        in_specs=[pl.BlockSpec((bn, f), lambda i: (i, 0))],
        out_specs=pl.BlockSpec((8, f), lambda i: (0, 0)),
        out_shape=jax.ShapeDtypeStruct((8, f), jnp.float32),
    )(x)


def _bn_leaky_body(x_ref, st_ref, g_ref, b_ref, o_ref, *, n, eps):
    mu = st_ref[0:1, :] / n
    var = st_ref[1:2, :] / n - mu * mu
    y = (x_ref[...] - mu) / jnp.sqrt(var + eps) * g_ref[...] + b_ref[...]
    o_ref[...] = _leaky(y)


def tc_bn_leaky(x, stats, g, b, eps=1e-5):
    n, f = x.shape
    bn = ROW_BLOCK
    body = functools.partial(_bn_leaky_body, n=float(n), eps=eps)
    return pl.pallas_call(
        body,
        grid=(n // bn,),
        in_specs=[
            pl.BlockSpec((bn, f), lambda i: (i, 0)),
            pl.BlockSpec((8, f), lambda i: (0, 0)),
            pl.BlockSpec((1, f), lambda i: (0, 0)),
            pl.BlockSpec((1, f), lambda i: (0, 0)),
        ],
        out_specs=pl.BlockSpec((bn, f), lambda i: (i, 0)),
        out_shape=jax.ShapeDtypeStruct((n, f), jnp.float32),
    )(x, stats, g.reshape(1, f), b.reshape(1, f))


def _logit_body(x_ref, w_ref, b_ref, o_ref):
    y = jnp.sum(x_ref[...] * w_ref[...], axis=1, keepdims=True) + b_ref[0]
    o_ref[...] = 1.0 / (1.0 + jnp.exp(-y))


def tc_logit_sigmoid(x, w, b):
    n, k = x.shape
    bn = ROW_BLOCK
    return pl.pallas_call(
        _logit_body,
        grid=(n // bn,),
        in_specs=[
            pl.BlockSpec((bn, k), lambda i: (i, 0)),
            pl.BlockSpec((1, k), lambda i: (0, 0)),
            pl.BlockSpec(memory_space=pltpu.SMEM),
        ],
        out_specs=pl.BlockSpec((bn, 1), lambda i: (i, 0)),
        out_shape=jax.ShapeDtypeStruct((n, 1), jnp.float32),
    )(x, w, b)


def _gcn_post_body(p_ref, s_ref, b_ref, o_ref):
    s = jnp.sum(s_ref[...], axis=1, keepdims=True)
    o_ref[...] = p_ref[...] * s + b_ref[...]


def tc_gcn_post(agg, dinv, b):
    n, f = agg.shape
    bn = ROW_BLOCK
    return pl.pallas_call(
        _gcn_post_body,
        grid=(n // bn,),
        in_specs=[
            pl.BlockSpec((bn, f), lambda i: (i, 0)),
            pl.BlockSpec((bn, 1), lambda i: (i, 0)),
            pl.BlockSpec((1, f), lambda i: (0, 0)),
        ],
        out_specs=pl.BlockSpec((bn, f), lambda i: (i, 0)),
        out_shape=jax.ShapeDtypeStruct((n, f), jnp.float32),
    )(agg, dinv.reshape(n, 1), b.reshape(1, f))


def _gat_post_body(num_ref, den_ref, b_ref, o_ref, *, heads):
    acc = None
    for h in range(heads):
        nh = num_ref[h]
        dh = jnp.sum(den_ref[h], axis=1, keepdims=True)
        t = nh / (dh + 1e-16)
        acc = t if acc is None else acc + t
    o_ref[...] = acc * (1.0 / heads) + b_ref[...]


def tc_gat_post(num, den_t, b):
    heads, n, fh = num.shape
    bn = ROW_BLOCK
    body = functools.partial(_gat_post_body, heads=heads)
    return pl.pallas_call(
        body,
        grid=(n // bn,),
        in_specs=[
            pl.BlockSpec((heads, bn, fh), lambda i: (0, i, 0)),
            pl.BlockSpec((heads, bn, 1), lambda i: (0, i, 0)),
            pl.BlockSpec((1, fh), lambda i: (0, 0)),
        ],
        out_specs=pl.BlockSpec((bn, fh), lambda i: (i, 0)),
        out_shape=jax.ShapeDtypeStruct((n, fh), jnp.float32),
    )(num, den_t, b.reshape(1, fh))


ROWS_PT = 624
ROWS_REM = N_NODES - 16 * ROWS_PT
ZREM = N_ACC - 16 * ROWS_PT


def _sc_mesh():
    return plsc.VectorSubcoreMesh(core_axis_name="c", subcore_axis_name="s",
                                  num_cores=1)


def _zero_acc(acc, buf, s):
    f = buf.shape[1]

    def zrow(r, _):
        buf[r] = jnp.zeros((f,), jnp.float32)
        return 0

    lax.fori_loop(0, CHUNK, zrow, 0)
    base = s * ROWS_PT
    for k in range(4):
        pltpu.sync_copy(buf, acc.at[pl.ds(base + 128 * k, 128)])
    pltpu.sync_copy(buf.at[pl.ds(0, 112)], acc.at[pl.ds(base + 512, 112)])

    @pl.when(s == 0)
    def _():
        pltpu.sync_copy(buf.at[pl.ds(0, ZREM)],
                        acc.at[pl.ds(16 * ROWS_PT, ZREM)])


def _copy_out(acc, out_hbm, s):
    pltpu.sync_copy(acc.at[pl.ds(s * ROWS_PT, ROWS_PT)],
                    out_hbm.at[pl.ds(s * ROWS_PT, ROWS_PT)])

    @pl.when(s == 0)
    def _():
        pltpu.sync_copy(acc.at[pl.ds(16 * ROWS_PT, ROWS_REM)],
                        out_hbm.at[pl.ds(16 * ROWS_PT, ROWS_REM)])


def _deg_body(dst_hbm, ones_hbm, out_hbm, acc, ones_v, didx):
    s = lax.axis_index("s")
    _zero_acc(acc, ones_v, s)
    pltpu.sync_copy(ones_hbm, ones_v)
    plsc.subcore_barrier()
    base0 = s * (TILE_CHUNKS * CHUNK)

    def chunk(i, _):
        base = base0 + i * CHUNK
        pltpu.sync_copy(dst_hbm.at[pl.ds(base, CHUNK)], didx.at[0])
        pltpu.sync_copy(ones_v, acc.at[didx.at[0]], add=True)
        return 0

    lax.fori_loop(0, TILE_CHUNKS, chunk, 0)
    plsc.subcore_barrier()
    _copy_out(acc, out_hbm, s)


def sc_degree(dstp):
    k = functools.partial(
        pl.kernel,
        out_type=jax.ShapeDtypeStruct((N_NODES, 128), jnp.float32),
        mesh=_sc_mesh(),
        scratch_types=[
            pltpu.VMEM_SHARED((N_ACC, 128), jnp.float32),
            pltpu.VMEM((CHUNK, 128), jnp.float32),
            pltpu.VMEM((1, CHUNK), jnp.int32),
        ],
    )(_deg_body)
    return k(dstp, jnp.ones((CHUNK, 128), jnp.float32))


def _gcn_body(g_hbm, src_hbm, dst_hbm, out_hbm, acc, rows, sidx, didx, sem):
    s = lax.axis_index("s")
    _zero_acc(acc, rows, s)
    plsc.subcore_barrier()
    base0 = s * (TILE_CHUNKS * CHUNK)

    def chunk(i, _):
        base = base0 + i * CHUNK
        pltpu.sync_copy(src_hbm.at[pl.ds(base, CHUNK)], sidx.at[0])
        pltpu.sync_copy(dst_hbm.at[pl.ds(base, CHUNK)], didx.at[0])
        pltpu.async_copy(g_hbm.at[sidx.at[0]], rows, sem).wait()
        pltpu.sync_copy(rows, acc.at[didx.at[0]], add=True)
        return 0

    lax.fori_loop(0, TILE_CHUNKS, chunk, 0)
    plsc.subcore_barrier()
    _copy_out(acc, out_hbm, s)


def sc_gcn_aggregate(g, srcp, dstp):
    k = functools.partial(
        pl.kernel,
        out_type=jax.ShapeDtypeStruct((N_NODES, 128), jnp.float32),
        mesh=_sc_mesh(),
        scratch_types=[
            pltpu.VMEM_SHARED((N_ACC, 128), jnp.float32),
            pltpu.VMEM((CHUNK, 128), jnp.float32),
            pltpu.VMEM((1, CHUNK), jnp.int32),
            pltpu.VMEM((1, CHUNK), jnp.int32),
            pltpu.SemaphoreType.DMA,
        ],
    )(_gcn_body)
    return k(g, srcp, dstp)


GAT_CHUNK = 64
GAT_TILE_CHUNKS = (TILE_CHUNKS * CHUNK) // GAT_CHUNK


def _gat_edge_body(as_hbm, ad_hbm, caps_hbm, src_hbm, dst_hbm,
                   w_hbm, den_hbm, acc, srows, drows, capv, wrows, w16,
                   sidx, didx, sem_a, sem_b):
    s = lax.axis_index("s")
    _zero_acc(acc, wrows, s)
    pltpu.sync_copy(caps_hbm, capv)
    plsc.subcore_barrier()
    base0 = s * (TILE_CHUNKS * CHUNK)

    def chunk(i, _):
        base = base0 + i * GAT_CHUNK
        pltpu.sync_copy(src_hbm.at[pl.ds(base, GAT_CHUNK)], sidx.at[0])
        pltpu.sync_copy(dst_hbm.at[pl.ds(base, GAT_CHUNK)], didx.at[0])
        ca = pltpu.async_copy(as_hbm.at[sidx.at[0]], srows, sem_a)
        cb = pltpu.async_copy(ad_hbm.at[didx.at[0]], drows, sem_b)
        ca.wait()
        cb.wait()
        cap_vec = capv[0]

        def edge(e, _):
            a = srows[e, pl.ds(0, 16)]
            b = drows[e, pl.ds(0, 16)]
            ee = a + b
            ee = jnp.where(ee >= 0, ee, 0.2 * ee)
            w = jnp.exp(ee - cap_vec)
            wrows[e, pl.ds(0, 16)] = w
            w16[e] = w
            return 0

        lax.fori_loop(0, GAT_CHUNK, edge, 0)
        pltpu.sync_copy(w16, w_hbm.at[pl.ds(base, GAT_CHUNK)])
        pltpu.sync_copy(wrows.at[pl.ds(0, GAT_CHUNK)],
                        acc.at[didx.at[0]], add=True)
        return 0

    lax.fori_loop(0, GAT_TILE_CHUNKS, chunk, 0)
    plsc.subcore_barrier()
    _copy_out(acc, den_hbm, s)


def sc_gat_edge(as128, ad128, caps, srcp, dstp):
    k = functools.partial(
        pl.kernel,
        out_type=(jax.ShapeDtypeStruct((EDGES_PAD, 16), jnp.float32),
                  jax.ShapeDtypeStruct((N_NODES, 128), jnp.float32)),
        mesh=_sc_mesh(),
        scratch_types=[
            pltpu.VMEM_SHARED((N_ACC, 128), jnp.float32),
            pltpu.VMEM((GAT_CHUNK, 128), jnp.float32),
            pltpu.VMEM((GAT_CHUNK, 128), jnp.float32),
            pltpu.VMEM((1, 16), jnp.float32),
            pltpu.VMEM((CHUNK, 128), jnp.float32),
            pltpu.VMEM((GAT_CHUNK, 16), jnp.float32),
            pltpu.VMEM((1, GAT_CHUNK), jnp.int32),
            pltpu.VMEM((1, GAT_CHUNK), jnp.int32),
            pltpu.SemaphoreType.DMA,
            pltpu.SemaphoreType.DMA,
        ],
    )(_gat_edge_body)
    return k(as128, ad128, caps, srcp, dstp)


def _gat_agg_body(tab_hbm, src_hbm, dst_hbm, w_hbm, out_hbm,
                  acc, rows, wrows, sidx, didx, sem, *, head):
    s = lax.axis_index("s")
    _zero_acc(acc, rows, s)
    plsc.subcore_barrier()
    base0 = s * (TILE_CHUNKS * CHUNK)

    def chunk(i, _):
        base = base0 + i * CHUNK
        pltpu.sync_copy(src_hbm.at[pl.ds(base, CHUNK)], sidx.at[0])
        pltpu.sync_copy(dst_hbm.at[pl.ds(base, CHUNK)], didx.at[0])
        pltpu.async_copy(tab_hbm.at[sidx.at[0]], rows, sem).wait()
        pltpu.sync_copy(w_hbm.at[pl.ds(base, CHUNK)], wrows)

        def edge(e, _):
            w = wrows[e][head]
            for q in range(8):
                rows[e, pl.ds(q * 16, 16)] = rows[e, pl.ds(q * 16, 16)] * w
            return 0

        lax.fori_loop(0, CHUNK, edge, 0)
        pltpu.sync_copy(rows, acc.at[didx.at[0]], add=True)
        return 0

    lax.fori_loop(0, TILE_CHUNKS, chunk, 0)
    plsc.subcore_barrier()
    _copy_out(acc, out_hbm, s)


def sc_gat_aggregate(tab, srcp, dstp, w_all, head):
    body = functools.partial(_gat_agg_body, head=head)
    k = functools.partial(
        pl.kernel,
        out_type=jax.ShapeDtypeStruct((N_NODES, 128), jnp.float32),
        mesh=_sc_mesh(),
        scratch_types=[
            pltpu.VMEM_SHARED((N_ACC, 128), jnp.float32),
            pltpu.VMEM((CHUNK, 128), jnp.float32),
            pltpu.VMEM((CHUNK, 16), jnp.float32),
            pltpu.VMEM((1, CHUNK), jnp.int32),
            pltpu.VMEM((1, CHUNK), jnp.int32),
            pltpu.SemaphoreType.DMA,
        ],
    )(body)
    return k(tab, srcp, dstp, w_all)


def _dinv_body(p_ref, o_ref):
    d = jnp.sum(p_ref[...], axis=1, keepdims=True) * (1.0 / 128.0)
    o_ref[...] = lax.rsqrt(d)


def tc_dinv(deg):
    n, _ = deg.shape
    bn = ROW_BLOCK
    return pl.pallas_call(
        _dinv_body,
        grid=(n // bn,),
        in_specs=[pl.BlockSpec((bn, 128), lambda i: (i, 0))],
        out_specs=pl.BlockSpec((bn, 1), lambda i: (i, 0)),
        out_shape=jax.ShapeDtypeStruct((n, 1), jnp.float32),
    )(deg)


def _caps_body(a_ref, d_ref, o_ref, m_ref, *, nsteps):
    i = pl.program_id(0)

    @pl.when(i == 0)
    def _():
        m_ref[...] = jnp.full_like(m_ref, -1e30)

    amax = jnp.max(a_ref[...], axis=0, keepdims=True)
    dmax = jnp.max(d_ref[...], axis=0, keepdims=True)
    z = jnp.zeros((6, amax.shape[1]), jnp.float32) - 1e30
    upd = jnp.concatenate([amax, dmax, z], axis=0)
    m_ref[...] = jnp.maximum(m_ref[...], upd)

    @pl.when(i == nsteps - 1)
    def _():
        c = m_ref[0:1, 0:16] + m_ref[1:2, 0:16]
        c = jnp.where(c >= 0, c, 0.2 * c)
        li = lax.broadcasted_iota(jnp.int32, (1, 16), 1)
        o_ref[...] = jnp.where(li < HEADS, c, 1e30)


def tc_caps(as128, ad128):
    n, f = as128.shape
    bn = ROW_BLOCK
    body = functools.partial(_caps_body, nsteps=n // bn)
    return pl.pallas_call(
        body,
        grid=(n // bn,),
        in_specs=[
            pl.BlockSpec((bn, f), lambda i: (i, 0)),
            pl.BlockSpec((bn, f), lambda i: (i, 0)),
        ],
        out_specs=pl.BlockSpec((1, 16), lambda i: (0, 0)),
        out_shape=jax.ShapeDtypeStruct((1, 16), jnp.float32),
        scratch_shapes=[pltpu.VMEM((8, f), jnp.float32)],
    )(as128, ad128)


def _den_t_body(d_ref, o_ref):
    d = d_ref[...]
    li = lax.broadcasted_iota(jnp.int32, d.shape, 1)
    parts = []
    for h in range(HEADS):
        sel = jnp.where(li == h, d, 0.0)
        parts.append(jnp.sum(sel, axis=1, keepdims=True)[None])
    o_ref[...] = jnp.concatenate(parts, axis=0)


def tc_den_t(den):
    n, _ = den.shape
    bn = ROW_BLOCK
    return pl.pallas_call(
        _den_t_body,
        grid=(n // bn,),
        in_specs=[pl.BlockSpec((bn, 128), lambda i: (i, 0))],
        out_specs=pl.BlockSpec((HEADS, bn, 1), lambda i: (0, i, 0)),
        out_shape=jax.ShapeDtypeStruct((HEADS, n, 1), jnp.float32),
    )(den)


def _gcn_layer(x, srcp, dstp, w, b, dinv, n):
    outs = []
    for c0 in range(0, w.shape[0], 128):
        wb = w[c0:c0 + 128]
        bb = b[c0:c0 + 128]
        fb = wb.shape[0]
        if fb < 128:
            wb = jnp.pad(wb, ((0, 128 - fb), (0, 0)))
            bb = jnp.pad(bb, (0, 128 - fb))
        g = tc_linear(x, wb, jnp.zeros((128,), jnp.float32), row_scale=dinv)
        agg = sc_gcn_aggregate(g, srcp, dstp)
        o = tc_gcn_post(agg, dinv, bb)
        outs.append(o[:, :fb] if fb < 128 else o)
    return outs[0] if len(outs) == 1 else jnp.concatenate(outs, axis=1)


def _gat_layer(x, srcp, dstp, p, pref, n):
    w = p[pref + '_w']
    fh = w.shape[0] // HEADS
    a_s, a_d = p[pref + '_as'], p[pref + '_ad']
    as_mat = jax.scipy.linalg.block_diag(*[a_s[k:k + 1, :] for k in range(HEADS)])
    ad_mat = jax.scipy.linalg.block_diag(*[a_d[k:k + 1, :] for k in range(HEADS)])
    asm128 = jnp.pad(as_mat @ w, ((0, 128 - HEADS), (0, 0)))
    adm128 = jnp.pad(ad_mat @ w, ((0, 128 - HEADS), (0, 0)))
    z128 = jnp.zeros((128,), jnp.float32)
    as128 = tc_linear(x, asm128, z128)
    ad128 = tc_linear(x, adm128, z128)
    caps = tc_caps(as128, ad128)
    as128 = jnp.pad(as128, ((0, N_ACC - n), (0, 0)))
    ad128 = jnp.pad(ad128, ((0, N_ACC - n), (0, 0)))
    w_all, den = sc_gat_edge(as128, ad128, caps, srcp, dstp)

    num_heads = []
    for h in range(HEADS):
        blocks = []
        for c0 in range(0, fh, 128):
            wb = w[h * fh + c0:h * fh + min(c0 + 128, fh)]
            fb = wb.shape[0]
            if fb < 128:
                wb = jnp.pad(wb, ((0, 128 - fb), (0, 0)))
            tab = tc_linear(x, wb, jnp.zeros((128,), jnp.float32))
            part = sc_gat_aggregate(tab, srcp, dstp, w_all, h)
            blocks.append(part[:, :fb] if fb < 128 else part)
        num_heads.append(blocks[0] if len(blocks) == 1
                         else jnp.concatenate(blocks, axis=1))
    num = jnp.stack(num_heads)
    den_t = tc_den_t(den)
    return tc_gat_post(num, den_t, p[pref + '_b'])


def kernel(x_in, edge_index, params):
    p = params
    n = x_in.shape[0]
    loop = jnp.arange(n, dtype=edge_index.dtype)
    src = jnp.concatenate([edge_index[0], loop])
    dst = jnp.concatenate([edge_index[1], loop])

    pad = EDGES_PAD - src.shape[0]
    srcp = jnp.concatenate([src, jnp.zeros((pad,), jnp.int32)])
    dstp = jnp.concatenate([dst, jnp.full((pad,), n, jnp.int32)])

    dinv = tc_dinv(sc_degree(dstp))

    x = tc_linear(x_in, p['nn1_w1'], p['nn1_b1'], act=_softplus)
    x = tc_linear(x, p['nn1_w2'], p['nn1_b2'], act=_softplus)
    x = tc_linear(x, p['nn1_w3'], p['nn1_b3'])
    x = tc_bn_leaky(x, tc_colstats(x), p['bn0_g'], p['bn0_b'])

    x1 = _gcn_layer(x, srcp, dstp, p['gcn1_w'], p['gcn1_b'], dinv, n)
    x1 = _gat_layer(x1, srcp, dstp, p, 'gat1', n)
    x1 = tc_bn_leaky(x1, tc_colstats(x1), p['bn1_g'], p['bn1_b'])

    skip1 = jnp.concatenate([x, x1], axis=1)
    x2 = _gcn_layer(skip1, srcp, dstp, p['gcn2_w'], p['gcn2_b'], dinv, n)
    x2 = _gat_layer(x2, srcp, dstp, p, 'gat2', n)
    x2 = tc_bn_leaky(x2, tc_colstats(x2), p['bn2_g'], p['bn2_b'])

    skip2 = jnp.concatenate([x1, x2], axis=1)
    x3 = _gcn_layer(skip2, srcp, dstp, p['gcn3_w'], p['gcn3_b'], dinv, n)
    x3 = _gat_layer(x3, srcp, dstp, p, 'gat3', n)
    x3 = tc_bn_leaky(x3, tc_colstats(x3), p['bn3_g'], p['bn3_b'])

    xf = jnp.concatenate([x, x3], axis=1)
    hh = tc_linear(xf, p['np_w1'], p['np_b1'], act=_softplus)
    hh = tc_linear(hh, p['np_w2'], p['np_b2'], act=_softplus)
    hh = tc_linear(hh, p['np_w3'], p['np_b3'], act=_softplus)
    probs = tc_logit_sigmoid(hh, p['np_w4'], p['np_b4'])
    return xf, probs


# speedup vs baseline: 4.8069x; 1.0924x over previous
import functools

import jax
import jax.numpy as jnp
from jax import lax
from jax.experimental import pallas as pl
from jax.experimental.pallas import tpu as pltpu
from jax.experimental.pallas import tpu_sc as plsc

HEADS = 5
ROW_BLOCK = 2000

N_NODES = 10000
N_ACC = N_NODES + 16
NUM_TILES = 32
CHUNK = 128
TILE_CHUNKS = 164
EDGES_PAD = 16 * TILE_CHUNKS * CHUNK


def _leaky(x, s=0.01):
    return jnp.where(x >= 0, x, s * x)


def _softplus(x):
    return jnp.maximum(x, 0.0) + jnp.log1p(jnp.exp(-jnp.abs(x)))


def _linear_body(x_ref, w_ref, b_ref, o_ref, *, act):
    y = jnp.dot(x_ref[...], w_ref[...].T, preferred_element_type=jnp.float32)
    y = y + b_ref[...]
    if act is not None:
        y = act(y)
    o_ref[...] = y


def _linear_scaled_body(x_ref, w_ref, b_ref, s_ref, o_ref, *, act):
    y = jnp.dot(x_ref[...], w_ref[...].T, preferred_element_type=jnp.float32)
    y = y + b_ref[...]
    if act is not None:
        y = act(y)
    s = jnp.sum(s_ref[...], axis=1, keepdims=True)
    o_ref[...] = y * s


def tc_linear(x, w, b, act=None, row_scale=None):
    n, k = x.shape
    f = w.shape[0]
    bn = ROW_BLOCK
    assert n % bn == 0
    b2 = b.reshape(1, f)
    if row_scale is None:
        body = functools.partial(_linear_body, act=act)
        return pl.pallas_call(
            body,
            grid=(n // bn,),
            in_specs=[
                pl.BlockSpec((bn, k), lambda i: (i, 0)),
                pl.BlockSpec((f, k), lambda i: (0, 0)),
                pl.BlockSpec((1, f), lambda i: (0, 0)),
            ],
            out_specs=pl.BlockSpec((bn, f), lambda i: (i, 0)),
            out_shape=jax.ShapeDtypeStruct((n, f), jnp.float32),
        )(x, w, b2)
    body = functools.partial(_linear_scaled_body, act=act)
    return pl.pallas_call(
        body,
        grid=(n // bn,),
        in_specs=[
            pl.BlockSpec((bn, k), lambda i: (i, 0)),
            pl.BlockSpec((f, k), lambda i: (0, 0)),
            pl.BlockSpec((1, f), lambda i: (0, 0)),
            pl.BlockSpec((bn, 1), lambda i: (i, 0)),
        ],
        out_specs=pl.BlockSpec((bn, f), lambda i: (i, 0)),
        out_shape=jax.ShapeDtypeStruct((n, f), jnp.float32),
    )(x, w, b2, row_scale.reshape(n, 1))


def _stats_body(x_ref, o_ref):
    i = pl.program_id(0)

    @pl.when(i == 0)
    def _():
        o_ref[...] = jnp.zeros_like(o_ref)

    xx = x_ref[...]
    s = jnp.sum(xx, axis=0, keepdims=True)
    ss = jnp.sum(xx * xx, axis=0, keepdims=True)
    pad = jnp.zeros((6, xx.shape[1]), jnp.float32)
    o_ref[...] += jnp.concatenate([s, ss, pad], axis=0)


def tc_colstats(x):
    n, f = x.shape
    bn = ROW_BLOCK
    return pl.pallas_call(
        _stats_body,
        grid=(n // bn,),
        in_specs=[pl.BlockSpec((bn, f), lambda i: (i, 0))],
        out_specs=pl.BlockSpec((8, f), lambda i: (0, 0)),
        out_shape=jax.ShapeDtypeStruct((8, f), jnp.float32),
    )(x)


def _bn_leaky_body(x_ref, st_ref, g_ref, b_ref, o_ref, *, n, eps):
    mu = st_ref[0:1, :] / n
    var = st_ref[1:2, :] / n - mu * mu
    y = (x_ref[...] - mu) / jnp.sqrt(var + eps) * g_ref[...] + b_ref[...]
    o_ref[...] = _leaky(y)


def tc_bn_leaky(x, stats, g, b, eps=1e-5):
    n, f = x.shape
    bn = ROW_BLOCK
    body = functools.partial(_bn_leaky_body, n=float(n), eps=eps)
    return pl.pallas_call(
        body,
        grid=(n // bn,),
        in_specs=[
            pl.BlockSpec((bn, f), lambda i: (i, 0)),
            pl.BlockSpec((8, f), lambda i: (0, 0)),
            pl.BlockSpec((1, f), lambda i: (0, 0)),
            pl.BlockSpec((1, f), lambda i: (0, 0)),
        ],
        out_specs=pl.BlockSpec((bn, f), lambda i: (i, 0)),
        out_shape=jax.ShapeDtypeStruct((n, f), jnp.float32),
    )(x, stats, g.reshape(1, f), b.reshape(1, f))


def _logit_body(x_ref, w_ref, b_ref, o_ref):
    y = jnp.sum(x_ref[...] * w_ref[...], axis=1, keepdims=True) + b_ref[0]
    o_ref[...] = 1.0 / (1.0 + jnp.exp(-y))


def tc_logit_sigmoid(x, w, b):
    n, k = x.shape
    bn = ROW_BLOCK
    return pl.pallas_call(
        _logit_body,
        grid=(n // bn,),
        in_specs=[
            pl.BlockSpec((bn, k), lambda i: (i, 0)),
            pl.BlockSpec((1, k), lambda i: (0, 0)),
            pl.BlockSpec(memory_space=pltpu.SMEM),
        ],
        out_specs=pl.BlockSpec((bn, 1), lambda i: (i, 0)),
        out_shape=jax.ShapeDtypeStruct((n, 1), jnp.float32),
    )(x, w, b)


def _gcn_post_body(p_ref, s_ref, b_ref, o_ref):
    s = jnp.sum(s_ref[...], axis=1, keepdims=True)
    o_ref[...] = p_ref[...] * s + b_ref[...]


def tc_gcn_post(agg, dinv, b):
    n, f = agg.shape
    bn = ROW_BLOCK
    return pl.pallas_call(
        _gcn_post_body,
        grid=(n // bn,),
        in_specs=[
            pl.BlockSpec((bn, f), lambda i: (i, 0)),
            pl.BlockSpec((bn, 1), lambda i: (i, 0)),
            pl.BlockSpec((1, f), lambda i: (0, 0)),
        ],
        out_specs=pl.BlockSpec((bn, f), lambda i: (i, 0)),
        out_shape=jax.ShapeDtypeStruct((n, f), jnp.float32),
    )(agg, dinv.reshape(n, 1), b.reshape(1, f))


def _gat_post_body(num_ref, den_ref, b_ref, o_ref, *, heads):
    acc = None
    for h in range(heads):
        nh = num_ref[h]
        dh = jnp.sum(den_ref[h], axis=1, keepdims=True)
        t = nh / (dh + 1e-16)
        acc = t if acc is None else acc + t
    o_ref[...] = acc * (1.0 / heads) + b_ref[...]


def tc_gat_post(num, den_t, b):
    heads, n, fh = num.shape
    bn = ROW_BLOCK
    body = functools.partial(_gat_post_body, heads=heads)
    return pl.pallas_call(
        body,
        grid=(n // bn,),
        in_specs=[
            pl.BlockSpec((heads, bn, fh), lambda i: (0, i, 0)),
            pl.BlockSpec((heads, bn, 1), lambda i: (0, i, 0)),
            pl.BlockSpec((1, fh), lambda i: (0, 0)),
        ],
        out_specs=pl.BlockSpec((bn, fh), lambda i: (i, 0)),
        out_shape=jax.ShapeDtypeStruct((n, fh), jnp.float32),
    )(num, den_t, b.reshape(1, fh))


ROWS_PT = 624
ROWS_REM = N_NODES - 16 * ROWS_PT
ZREM = N_ACC - 16 * ROWS_PT


def _sc_mesh():
    return plsc.VectorSubcoreMesh(core_axis_name="c", subcore_axis_name="s",
                                  num_cores=1)


def _zero_acc(acc, buf, s):
    f = buf.shape[1]

    def zrow(r, _):
        buf[r] = jnp.zeros((f,), jnp.float32)
        return 0

    lax.fori_loop(0, CHUNK, zrow, 0)
    base = s * ROWS_PT
    for k in range(4):
        pltpu.sync_copy(buf, acc.at[pl.ds(base + 128 * k, 128)])
    pltpu.sync_copy(buf.at[pl.ds(0, 112)], acc.at[pl.ds(base + 512, 112)])

    @pl.when(s == 0)
    def _():
        pltpu.sync_copy(buf.at[pl.ds(0, ZREM)],
                        acc.at[pl.ds(16 * ROWS_PT, ZREM)])


def _copy_out(acc, out_hbm, s):
    pltpu.sync_copy(acc.at[pl.ds(s * ROWS_PT, ROWS_PT)],
                    out_hbm.at[pl.ds(s * ROWS_PT, ROWS_PT)])

    @pl.when(s == 0)
    def _():
        pltpu.sync_copy(acc.at[pl.ds(16 * ROWS_PT, ROWS_REM)],
                        out_hbm.at[pl.ds(16 * ROWS_PT, ROWS_REM)])


def _deg_body(dst_hbm, ones_hbm, out_hbm, acc, ones_v, didx):
    s = lax.axis_index("s")
    _zero_acc(acc, ones_v, s)
    pltpu.sync_copy(ones_hbm, ones_v)
    plsc.subcore_barrier()
    base0 = s * (TILE_CHUNKS * CHUNK)

    def chunk(i, _):
        base = base0 + i * CHUNK
        pltpu.sync_copy(dst_hbm.at[pl.ds(base, CHUNK)], didx.at[0])
        pltpu.sync_copy(ones_v, acc.at[didx.at[0]], add=True)
        return 0

    lax.fori_loop(0, TILE_CHUNKS, chunk, 0)
    plsc.subcore_barrier()
    _copy_out(acc, out_hbm, s)


def sc_degree(dstp):
    k = functools.partial(
        pl.kernel,
        out_type=jax.ShapeDtypeStruct((N_NODES, 128), jnp.float32),
        mesh=_sc_mesh(),
        scratch_types=[
            pltpu.VMEM_SHARED((N_ACC, 128), jnp.float32),
            pltpu.VMEM((CHUNK, 128), jnp.float32),
            pltpu.VMEM((1, CHUNK), jnp.int32),
        ],
    )(_deg_body)
    return k(dstp, jnp.ones((CHUNK, 128), jnp.float32))


def _gcn_body(g_hbm, src_hbm, dst_hbm, out_hbm, acc, rows_a, rows_b,
              sidx, didx, sem_a, sem_b):
    s = lax.axis_index("s")
    _zero_acc(acc, rows_a, s)
    plsc.subcore_barrier()
    base0 = s * (TILE_CHUNKS * CHUNK)

    def pair(k, _):
        base = base0 + k * (2 * CHUNK)
        pltpu.sync_copy(src_hbm.at[pl.ds(base, CHUNK)], sidx.at[0])
        pltpu.sync_copy(src_hbm.at[pl.ds(base + CHUNK, CHUNK)], sidx.at[1])
        pltpu.sync_copy(dst_hbm.at[pl.ds(base, CHUNK)], didx.at[0])
        pltpu.sync_copy(dst_hbm.at[pl.ds(base + CHUNK, CHUNK)], didx.at[1])
        ga = pltpu.async_copy(g_hbm.at[sidx.at[0]], rows_a, sem_a)
        gb = pltpu.async_copy(g_hbm.at[sidx.at[1]], rows_b, sem_b)
        ga.wait()
        pltpu.sync_copy(rows_a, acc.at[didx.at[0]], add=True)
        gb.wait()
        pltpu.sync_copy(rows_b, acc.at[didx.at[1]], add=True)
        return 0

    lax.fori_loop(0, TILE_CHUNKS // 2, pair, 0)
    plsc.subcore_barrier()
    _copy_out(acc, out_hbm, s)


def sc_gcn_aggregate(g, srcp, dstp):
    k = functools.partial(
        pl.kernel,
        out_type=jax.ShapeDtypeStruct((N_NODES, 128), jnp.float32),
        mesh=_sc_mesh(),
        scratch_types=[
            pltpu.VMEM_SHARED((N_ACC, 128), jnp.float32),
            pltpu.VMEM((CHUNK, 128), jnp.float32),
            pltpu.VMEM((CHUNK, 128), jnp.float32),
            pltpu.VMEM((2, CHUNK), jnp.int32),
            pltpu.VMEM((2, CHUNK), jnp.int32),
            pltpu.SemaphoreType.DMA,
            pltpu.SemaphoreType.DMA,
        ],
    )(_gcn_body)
    return k(g, srcp, dstp)


GAT_CHUNK = 64
GAT_TILE_CHUNKS = (TILE_CHUNKS * CHUNK) // GAT_CHUNK


def _gat_edge_body(as_hbm, ad_hbm, caps_hbm, src_hbm, dst_hbm,
                   w_hbm, den_hbm, acc, srows, drows, capv, wrows, w16,
                   sidx, didx, sem_a, sem_b):
    s = lax.axis_index("s")
    _zero_acc(acc, wrows, s)
    pltpu.sync_copy(caps_hbm, capv)
    plsc.subcore_barrier()
    base0 = s * (TILE_CHUNKS * CHUNK)

    def chunk(i, _):
        base = base0 + i * GAT_CHUNK
        pltpu.sync_copy(src_hbm.at[pl.ds(base, GAT_CHUNK)], sidx.at[0])
        pltpu.sync_copy(dst_hbm.at[pl.ds(base, GAT_CHUNK)], didx.at[0])
        ca = pltpu.async_copy(as_hbm.at[sidx.at[0]], srows, sem_a)
        cb = pltpu.async_copy(ad_hbm.at[didx.at[0]], drows, sem_b)
        ca.wait()
        cb.wait()
        cap_vec = capv[0]

        def edge(e, _):
            a = srows[e, pl.ds(0, 16)]
            b = drows[e, pl.ds(0, 16)]
            ee = a + b
            ee = jnp.where(ee >= 0, ee, 0.2 * ee)
            w = jnp.exp(ee - cap_vec)
            wrows[e, pl.ds(0, 16)] = w
            w16[pl.ds(e * 16, 16)] = w
            return 0

        lax.fori_loop(0, GAT_CHUNK, edge, 0)
        pltpu.sync_copy(w16, w_hbm.at[pl.ds(base * 16, GAT_CHUNK * 16)])
        pltpu.sync_copy(wrows.at[pl.ds(0, GAT_CHUNK)],
                        acc.at[didx.at[0]], add=True)
        return 0

    lax.fori_loop(0, GAT_TILE_CHUNKS, chunk, 0)
    plsc.subcore_barrier()
    _copy_out(acc, den_hbm, s)


def sc_gat_edge(as128, ad128, caps, srcp, dstp):
    k = functools.partial(
        pl.kernel,
        out_type=(jax.ShapeDtypeStruct((EDGES_PAD * 16,), jnp.float32),
                  jax.ShapeDtypeStruct((N_NODES, 128), jnp.float32)),
        mesh=_sc_mesh(),
        scratch_types=[
            pltpu.VMEM_SHARED((N_ACC, 128), jnp.float32),
            pltpu.VMEM((GAT_CHUNK, 128), jnp.float32),
            pltpu.VMEM((GAT_CHUNK, 128), jnp.float32),
            pltpu.VMEM((1, 16), jnp.float32),
            pltpu.VMEM((CHUNK, 128), jnp.float32),
            pltpu.VMEM((GAT_CHUNK * 16,), jnp.float32),
            pltpu.VMEM((1, GAT_CHUNK), jnp.int32),
            pltpu.VMEM((1, GAT_CHUNK), jnp.int32),
            pltpu.SemaphoreType.DMA,
            pltpu.SemaphoreType.DMA,
        ],
    )(_gat_edge_body)
    return k(as128, ad128, caps, srcp, dstp)


def _gat_agg_scale(rows, wrows, head):
    def edge(e, _):
        w = wrows[pl.ds(e * 16, 16)][head]
        for q in range(8):
            rows[e, pl.ds(q * 16, 16)] = rows[e, pl.ds(q * 16, 16)] * w
        return 0

    lax.fori_loop(0, CHUNK, edge, 0)


def _gat_agg_body(tab_hbm, src_hbm, dst_hbm, w_hbm, out_hbm,
                  acc, rows_a, rows_b, wrows_a, wrows_b, sidx, didx,
                  sem_a, sem_b, *, head):
    s = lax.axis_index("s")
    _zero_acc(acc, rows_a, s)
    plsc.subcore_barrier()
    base0 = s * (TILE_CHUNKS * CHUNK)

    def pair(k, _):
        base = base0 + k * (2 * CHUNK)
        pltpu.sync_copy(src_hbm.at[pl.ds(base, CHUNK)], sidx.at[0])
        pltpu.sync_copy(src_hbm.at[pl.ds(base + CHUNK, CHUNK)], sidx.at[1])
        pltpu.sync_copy(dst_hbm.at[pl.ds(base, CHUNK)], didx.at[0])
        pltpu.sync_copy(dst_hbm.at[pl.ds(base + CHUNK, CHUNK)], didx.at[1])
        ga = pltpu.async_copy(tab_hbm.at[sidx.at[0]], rows_a, sem_a)
        gb = pltpu.async_copy(tab_hbm.at[sidx.at[1]], rows_b, sem_b)
        pltpu.sync_copy(w_hbm.at[pl.ds(base * 16, CHUNK * 16)], wrows_a)
        pltpu.sync_copy(w_hbm.at[pl.ds((base + CHUNK) * 16, CHUNK * 16)],
                        wrows_b)
        ga.wait()
        _gat_agg_scale(rows_a, wrows_a, head)
        pltpu.sync_copy(rows_a, acc.at[didx.at[0]], add=True)
        gb.wait()
        _gat_agg_scale(rows_b, wrows_b, head)
        pltpu.sync_copy(rows_b, acc.at[didx.at[1]], add=True)
        return 0

    lax.fori_loop(0, TILE_CHUNKS // 2, pair, 0)
    plsc.subcore_barrier()
    _copy_out(acc, out_hbm, s)


def sc_gat_aggregate(tab, srcp, dstp, w_all, head):
    body = functools.partial(_gat_agg_body, head=head)
    k = functools.partial(
        pl.kernel,
        out_type=jax.ShapeDtypeStruct((N_NODES, 128), jnp.float32),
        mesh=_sc_mesh(),
        scratch_types=[
            pltpu.VMEM_SHARED((N_ACC, 128), jnp.float32),
            pltpu.VMEM((CHUNK, 128), jnp.float32),
            pltpu.VMEM((CHUNK, 128), jnp.float32),
            pltpu.VMEM((CHUNK * 16,), jnp.float32),
            pltpu.VMEM((CHUNK * 16,), jnp.float32),
            pltpu.VMEM((2, CHUNK), jnp.int32),
            pltpu.VMEM((2, CHUNK), jnp.int32),
            pltpu.SemaphoreType.DMA,
            pltpu.SemaphoreType.DMA,
        ],
    )(body)
    return k(tab, srcp, dstp, w_all)


def _dinv_body(p_ref, o_ref):
    d = jnp.sum(p_ref[...], axis=1, keepdims=True) * (1.0 / 128.0)
    o_ref[...] = lax.rsqrt(d)


def tc_dinv(deg):
    n, _ = deg.shape
    bn = ROW_BLOCK
    return pl.pallas_call(
        _dinv_body,
        grid=(n // bn,),
        in_specs=[pl.BlockSpec((bn, 128), lambda i: (i, 0))],
        out_specs=pl.BlockSpec((bn, 1), lambda i: (i, 0)),
        out_shape=jax.ShapeDtypeStruct((n, 1), jnp.float32),
    )(deg)


def _caps_body(a_ref, d_ref, o_ref, m_ref, *, nsteps):
    i = pl.program_id(0)

    @pl.when(i == 0)
    def _():
        m_ref[...] = jnp.full_like(m_ref, -1e30)

    amax = jnp.max(a_ref[...], axis=0, keepdims=True)
    dmax = jnp.max(d_ref[...], axis=0, keepdims=True)
    z = jnp.zeros((6, amax.shape[1]), jnp.float32) - 1e30
    upd = jnp.concatenate([amax, dmax, z], axis=0)
    m_ref[...] = jnp.maximum(m_ref[...], upd)

    @pl.when(i == nsteps - 1)
    def _():
        c = m_ref[0:1, 0:16] + m_ref[1:2, 0:16]
        c = jnp.where(c >= 0, c, 0.2 * c)
        li = lax.broadcasted_iota(jnp.int32, (1, 16), 1)
        o_ref[...] = jnp.where(li < HEADS, c, 1e30)


def tc_caps(as128, ad128):
    n, f = as128.shape
    bn = ROW_BLOCK
    body = functools.partial(_caps_body, nsteps=n // bn)
    return pl.pallas_call(
        body,
        grid=(n // bn,),
        in_specs=[
            pl.BlockSpec((bn, f), lambda i: (i, 0)),
            pl.BlockSpec((bn, f), lambda i: (i, 0)),
        ],
        out_specs=pl.BlockSpec((1, 16), lambda i: (0, 0)),
        out_shape=jax.ShapeDtypeStruct((1, 16), jnp.float32),
        scratch_shapes=[pltpu.VMEM((8, f), jnp.float32)],
    )(as128, ad128)


def _den_t_body(d_ref, o_ref):
    d = d_ref[...]
    li = lax.broadcasted_iota(jnp.int32, d.shape, 1)
    parts = []
    for h in range(HEADS):
        sel = jnp.where(li == h, d, 0.0)
        parts.append(jnp.sum(sel, axis=1, keepdims=True)[None])
    o_ref[...] = jnp.concatenate(parts, axis=0)


def tc_den_t(den):
    n, _ = den.shape
    bn = ROW_BLOCK
    return pl.pallas_call(
        _den_t_body,
        grid=(n // bn,),
        in_specs=[pl.BlockSpec((bn, 128), lambda i: (i, 0))],
        out_specs=pl.BlockSpec((HEADS, bn, 1), lambda i: (0, i, 0)),
        out_shape=jax.ShapeDtypeStruct((HEADS, n, 1), jnp.float32),
    )(den)


def _gcn_layer(x, srcp, dstp, w, b, dinv, n):
    outs = []
    for c0 in range(0, w.shape[0], 128):
        wb = w[c0:c0 + 128]
        bb = b[c0:c0 + 128]
        fb = wb.shape[0]
        if fb < 128:
            wb = jnp.pad(wb, ((0, 128 - fb), (0, 0)))
            bb = jnp.pad(bb, (0, 128 - fb))
        g = tc_linear(x, wb, jnp.zeros((128,), jnp.float32), row_scale=dinv)
        agg = sc_gcn_aggregate(g, srcp, dstp)
        o = tc_gcn_post(agg, dinv, bb)
        outs.append(o[:, :fb] if fb < 128 else o)
    return outs[0] if len(outs) == 1 else jnp.concatenate(outs, axis=1)


def _gat_layer(x, srcp, dstp, p, pref, n):
    w = p[pref + '_w']
    fh = w.shape[0] // HEADS
    a_s, a_d = p[pref + '_as'], p[pref + '_ad']
    as_mat = jax.scipy.linalg.block_diag(*[a_s[k:k + 1, :] for k in range(HEADS)])
    ad_mat = jax.scipy.linalg.block_diag(*[a_d[k:k + 1, :] for k in range(HEADS)])
    asm128 = jnp.pad(as_mat @ w, ((0, 128 - HEADS), (0, 0)))
    adm128 = jnp.pad(ad_mat @ w, ((0, 128 - HEADS), (0, 0)))
    z128 = jnp.zeros((128,), jnp.float32)
    as128 = tc_linear(x, asm128, z128)
    ad128 = tc_linear(x, adm128, z128)
    caps = tc_caps(as128, ad128)
    as128 = jnp.pad(as128, ((0, N_ACC - n), (0, 0)))
    ad128 = jnp.pad(ad128, ((0, N_ACC - n), (0, 0)))
    w_all, den = sc_gat_edge(as128, ad128, caps, srcp, dstp)

    num_heads = []
    for h in range(HEADS):
        blocks = []
        for c0 in range(0, fh, 128):
            wb = w[h * fh + c0:h * fh + min(c0 + 128, fh)]
            fb = wb.shape[0]
            if fb < 128:
                wb = jnp.pad(wb, ((0, 128 - fb), (0, 0)))
            tab = tc_linear(x, wb, jnp.zeros((128,), jnp.float32))
            part = sc_gat_aggregate(tab, srcp, dstp, w_all, h)
            blocks.append(part[:, :fb] if fb < 128 else part)
        num_heads.append(blocks[0] if len(blocks) == 1
                         else jnp.concatenate(blocks, axis=1))
    num = jnp.stack(num_heads)
    den_t = tc_den_t(den)
    return tc_gat_post(num, den_t, p[pref + '_b'])


def kernel(x_in, edge_index, params):
    p = params
    n = x_in.shape[0]
    loop = jnp.arange(n, dtype=edge_index.dtype)
    src = jnp.concatenate([edge_index[0], loop])
    dst = jnp.concatenate([edge_index[1], loop])

    pad = EDGES_PAD - src.shape[0]
    srcp = jnp.concatenate([src, jnp.zeros((pad,), jnp.int32)])
    dstp = jnp.concatenate([dst, jnp.full((pad,), n, jnp.int32)])

    dinv = tc_dinv(sc_degree(dstp))

    x = tc_linear(x_in, p['nn1_w1'], p['nn1_b1'], act=_softplus)
    x = tc_linear(x, p['nn1_w2'], p['nn1_b2'], act=_softplus)
    x = tc_linear(x, p['nn1_w3'], p['nn1_b3'])
    x = tc_bn_leaky(x, tc_colstats(x), p['bn0_g'], p['bn0_b'])

    x1 = _gcn_layer(x, srcp, dstp, p['gcn1_w'], p['gcn1_b'], dinv, n)
    x1 = _gat_layer(x1, srcp, dstp, p, 'gat1', n)
    x1 = tc_bn_leaky(x1, tc_colstats(x1), p['bn1_g'], p['bn1_b'])

    skip1 = jnp.concatenate([x, x1], axis=1)
    x2 = _gcn_layer(skip1, srcp, dstp, p['gcn2_w'], p['gcn2_b'], dinv, n)
    x2 = _gat_layer(x2, srcp, dstp, p, 'gat2', n)
    x2 = tc_bn_leaky(x2, tc_colstats(x2), p['bn2_g'], p['bn2_b'])

    skip2 = jnp.concatenate([x1, x2], axis=1)
    x3 = _gcn_layer(skip2, srcp, dstp, p['gcn3_w'], p['gcn3_b'], dinv, n)
    x3 = _gat_layer(x3, srcp, dstp, p, 'gat3', n)
    x3 = tc_bn_leaky(x3, tc_colstats(x3), p['bn3_g'], p['bn3_b'])

    xf = jnp.concatenate([x, x3], axis=1)
    hh = tc_linear(xf, p['np_w1'], p['np_b1'], act=_softplus)
    hh = tc_linear(hh, p['np_w2'], p['np_b2'], act=_softplus)
    hh = tc_linear(hh, p['np_w3'], p['np_b3'], act=_softplus)
    probs = tc_logit_sigmoid(hh, p['np_w4'], p['np_b4'])
    return xf, probs
